# baseline (speedup 1.0000x reference)
import functools

import jax
import jax.numpy as jnp
from jax import lax
from jax.experimental import pallas as pl
from jax.experimental.pallas import tpu as pltpu
from jax.experimental.pallas import tpu_sc as plsc

N_NODES = 4096
N_EDGES = 131072
EPS = 1e-6

NS = 16
EPT = N_EDGES // NS
HALF = EPT // 2
NCHUNK = EPT // 16

_sc_mesh = plsc.VectorSubcoreMesh(core_axis_name="c", subcore_axis_name="s")
_sc_params = pltpu.CompilerParams(needs_layout_passes=False)


@functools.partial(
    pl.kernel,
    out_type=jax.ShapeDtypeStruct((4, N_EDGES), jnp.float32),
    mesh=_sc_mesh,
    scratch_types=[
        pltpu.VMEM((4, EPT), jnp.float32),
        pltpu.VMEM((EPT,), jnp.int32),
        pltpu.VMEM((4, EPT), jnp.float32),
        pltpu.VMEM((32, 512), jnp.float32),
        pltpu.VMEM((4, HALF), jnp.float32),
        pltpu.VMEM((2, 512), jnp.float32),
        pltpu.VMEM((2, 512), jnp.float32),
        pltpu.VMEM((16, 16), jnp.float32),
        pltpu.VMEM_SHARED((16, 32, 512), jnp.float32),
        pltpu.VMEM_SHARED((32, 512), jnp.float32),
    ],
    compiler_params=_sc_params,
)
def _edge_norm(edge_w_hbm, wbrd_hbm, dst_hbm, out_hbm,
               ew, dstb, eb, dn, wb_, tmp2, acc2, wbv, sh16, shfin):
    c = lax.axis_index("c")
    s = lax.axis_index("s")
    base = s * EPT

    pltpu.sync_copy(dst_hbm.at[pl.ds(base, EPT)], dstb)
    for j in range(4):
        pltpu.sync_copy(edge_w_hbm.at[j, pl.ds(base, EPT)], ew.at[j])
    pltpu.sync_copy(wbrd_hbm, wbv)

    def zero_row(i, _):
        for k2 in range(32):
            dn[i, pl.ds(k2 * 16, 16)] = jnp.zeros((16,), jnp.float32)
        return 0
    lax.fori_loop(0, 32, zero_row, 0)

    wrows = [wbv[k, :] for k in range(16)]

    def acc(i, _):
        off = i * 16
        dv = dstb[pl.ds(off, 16)]
        ewv = [ew[j, pl.ds(off, 16)] for j in range(4)]
        for p in range(4):
            ws = (wrows[4 * p] * ewv[0] + wrows[4 * p + 1] * ewv[1]
                  + wrows[4 * p + 2] * ewv[2] + wrows[4 * p + 3] * ewv[3])
            ev = jnp.exp(ws)
            eb[p, pl.ds(off, 16)] = ev
            plsc.addupdate_scatter(dn, [(dv >> 9) + 8 * p, dv & 511], ev)
        return 0
    lax.fori_loop(0, NCHUNK, acc, 0)

    pltpu.sync_copy(dn, sh16.at[s])
    plsc.subcore_barrier()
    for r in range(2):
        for k2 in range(32):
            acc2[r, pl.ds(k2 * 16, 16)] = jnp.zeros((16,), jnp.float32)
    for t2 in range(16):
        pltpu.sync_copy(sh16.at[t2, pl.ds(2 * s, 2)], tmp2)

        def radd(k, _):
            for r in range(2):
                acc2[r, pl.ds(k * 16, 16)] += tmp2[r, pl.ds(k * 16, 16)]
            return 0
        lax.fori_loop(0, 32, radd, 0)
    pltpu.sync_copy(acc2, shfin.at[pl.ds(2 * s, 2)])
    plsc.subcore_barrier()
    pltpu.sync_copy(shfin, dn)

    def wcomp(i, _):
        off = c * HALF + i * 16
        dv = dstb[pl.ds(off, 16)]
        for p in range(4):
            ev = eb[p, pl.ds(off, 16)]
            g = plsc.load_gather(dn, [(dv >> 9) + 8 * p, dv & 511])
            wb_[p, pl.ds(i * 16, 16)] = ev / (g + EPS)
        return 0
    lax.fori_loop(0, HALF // 16, wcomp, 0)

    for p in range(4):
        pltpu.sync_copy(wb_.at[p], out_hbm.at[p, pl.ds(base + c * HALF, HALF)])


WROWS = 16
NWIN = N_NODES // WROWS
WPT = NWIN // NS
CAP = 16384
SCHUNK = 1024


def _make_build_dense(x):
    @functools.partial(
        pl.kernel,
        out_type=jax.ShapeDtypeStruct((2 * N_NODES, N_NODES), jnp.float32),
        mesh=_sc_mesh,
        scratch_types=[
            pltpu.VMEM((SCHUNK,), jnp.int32),
            pltpu.VMEM((SCHUNK,), jnp.int32),
            pltpu.VMEM((SCHUNK,), jnp.float32),
            pltpu.VMEM((CAP,), jnp.int32),
            pltpu.VMEM((CAP,), jnp.float32),
            pltpu.VMEM((WROWS, N_NODES), jnp.float32),
        ],
        compiler_params=_sc_params,
    )
    def _build_pair(src_hbm, dst_hbm, w_hbm, dense_hbm,
                    ssrc, sdst, sw, kk, kw, slab):
        c = lax.axis_index("c")
        s = lax.axis_index("s")

        def stream(i, cnt):
            off = i * SCHUNK
            pltpu.sync_copy(src_hbm.at[pl.ds(off, SCHUNK)], ssrc)
            pltpu.sync_copy(dst_hbm.at[pl.ds(off, SCHUNK)], sdst)
            pltpu.sync_copy(w_hbm.at[x + 2 * c, pl.ds(off, SCHUNK)], sw)

            def chunk(j, cnt2):
                o2 = j * 16
                sv = ssrc[pl.ds(o2, 16)]
                dv = sdst[pl.ds(o2, 16)]
                key = sv * 4096 + dv
                m = ((sv >> 4) & 15) == s
                cc = jnp.minimum(cnt2, CAP - 16)
                plsc.store_compressed(kk.at[pl.ds(cc, 16)], key, mask=m)
                plsc.store_compressed(kw.at[pl.ds(cc, 16)], sw[pl.ds(o2, 16)], mask=m)
                return cc + jnp.sum(m.astype(jnp.int32))
            return lax.fori_loop(0, SCHUNK // 16, chunk, cnt)
        cnt = lax.fori_loop(0, N_EDGES // SCHUNK, stream, 0)
        nchunks = (cnt + 15) >> 4

        def sweep(k, _):
            wid = k * NS + s
            row0 = wid * WROWS

            def zero_row(i, _):
                for k2 in range(16):
                    slab[i >> 4, pl.ds(((i & 15) * 16 + k2) * 16, 16)] = (
                        jnp.zeros((16,), jnp.float32))
                return 0
            lax.fori_loop(0, WROWS * 16, zero_row, 0)

            def scan(i, _):
                o2 = i * 16
                kv = kk[pl.ds(o2, 16)]
                m = (kv >> 16) == wid
                wv = jnp.where(m, kw[pl.ds(o2, 16)], 0.0)
                plsc.addupdate_scatter(slab, [(kv >> 12) & 15, kv & 4095], wv)
                return 0
            lax.fori_loop(0, nchunks, scan, 0)

            pltpu.sync_copy(slab, dense_hbm.at[pl.ds(c * N_NODES + row0, WROWS)])
            return 0

        lax.fori_loop(0, WPT, sweep, 0)

    return _build_pair


_build_pair0 = _make_build_dense(0)
_build_pair1 = _make_build_dense(1)


BM = 512
BN = 2048


def _mm_body(a_ref, b_ref, o_ref):
    o_ref[...] = jnp.dot(a_ref[...], b_ref[...], preferred_element_type=jnp.float32)


@jax.jit
def _mm(ab):
    n = ab.shape[-1]
    grid = (n // BN, n // BM)
    return pl.pallas_call(
        _mm_body,
        grid=grid,
        in_specs=[
            pl.BlockSpec((BM, n), lambda j, i: (i, 0)),
            pl.BlockSpec((n, BN), lambda j, i: (1, j)),
        ],
        out_specs=pl.BlockSpec((BM, BN), lambda j, i: (i, j)),
        out_shape=jax.ShapeDtypeStruct((n, n), jnp.float32),
        compiler_params=pltpu.CompilerParams(
            dimension_semantics=("arbitrary", "arbitrary"),
        ),
    )(ab, ab)


def kernel(edge_w, W1, W2, edge_src, edge_dst):
    Wc = jnp.concatenate([W1, W2], axis=0).reshape(16)
    Wbrd = jnp.broadcast_to(Wc[:, None], (16, 16))
    w = _edge_norm(edge_w, Wbrd, edge_dst)
    d0 = _build_pair0(edge_src, edge_dst, w)
    C0 = _mm(d0.astype(jnp.bfloat16))
    d1 = _build_pair1(edge_src, edge_dst, w)
    C1 = _mm(d1.astype(jnp.bfloat16))
    C = jnp.stack([C0, C1])
    return (C, W1, W2, w[:2])


# speedup vs baseline: 1.1517x; 1.1517x over previous
import functools

import jax
import jax.numpy as jnp
from jax import lax
from jax.experimental import pallas as pl
from jax.experimental.pallas import tpu as pltpu
from jax.experimental.pallas import tpu_sc as plsc

N_NODES = 4096
N_EDGES = 131072
EPS = 1e-6

NS = 16
EPT = N_EDGES // NS
HALF = EPT // 2
NCHUNK = EPT // 16

_sc_mesh = plsc.VectorSubcoreMesh(core_axis_name="c", subcore_axis_name="s")
_sc_params = pltpu.CompilerParams(needs_layout_passes=False)


@functools.partial(
    pl.kernel,
    out_type=jax.ShapeDtypeStruct((4, N_EDGES), jnp.float32),
    mesh=_sc_mesh,
    scratch_types=[
        pltpu.VMEM((4, EPT), jnp.float32),
        pltpu.VMEM((EPT,), jnp.int32),
        pltpu.VMEM((4, EPT), jnp.float32),
        pltpu.VMEM((32, 512), jnp.float32),
        pltpu.VMEM((4, HALF), jnp.float32),
        pltpu.VMEM((2, 512), jnp.float32),
        pltpu.VMEM((2, 512), jnp.float32),
        pltpu.VMEM((16, 16), jnp.float32),
        pltpu.VMEM_SHARED((16, 32, 512), jnp.float32),
        pltpu.VMEM_SHARED((32, 512), jnp.float32),
    ],
    compiler_params=_sc_params,
)
def _edge_norm(edge_w_hbm, wbrd_hbm, dst_hbm, out_hbm,
               ew, dstb, eb, dn, wb_, tmp2, acc2, wbv, sh16, shfin):
    c = lax.axis_index("c")
    s = lax.axis_index("s")
    base = s * EPT

    pltpu.sync_copy(dst_hbm.at[pl.ds(base, EPT)], dstb)
    for j in range(4):
        pltpu.sync_copy(edge_w_hbm.at[j, pl.ds(base, EPT)], ew.at[j])
    pltpu.sync_copy(wbrd_hbm, wbv)

    def zero_row(i, _):
        for k2 in range(32):
            dn[i, pl.ds(k2 * 16, 16)] = jnp.zeros((16,), jnp.float32)
        return 0
    lax.fori_loop(0, 32, zero_row, 0)

    wrows = [wbv[k, :] for k in range(16)]

    def acc(i, _):
        off = i * 16
        dv = dstb[pl.ds(off, 16)]
        ewv = [ew[j, pl.ds(off, 16)] for j in range(4)]
        for p in range(4):
            ws = (wrows[4 * p] * ewv[0] + wrows[4 * p + 1] * ewv[1]
                  + wrows[4 * p + 2] * ewv[2] + wrows[4 * p + 3] * ewv[3])
            ev = jnp.exp(ws)
            eb[p, pl.ds(off, 16)] = ev
            plsc.addupdate_scatter(dn, [(dv >> 9) + 8 * p, dv & 511], ev)
        return 0
    lax.fori_loop(0, NCHUNK, acc, 0)

    pltpu.sync_copy(dn, sh16.at[s])
    plsc.subcore_barrier()
    for r in range(2):
        for k2 in range(32):
            acc2[r, pl.ds(k2 * 16, 16)] = jnp.zeros((16,), jnp.float32)
    for t2 in range(16):
        pltpu.sync_copy(sh16.at[t2, pl.ds(2 * s, 2)], tmp2)

        def radd(k, _):
            for r in range(2):
                acc2[r, pl.ds(k * 16, 16)] += tmp2[r, pl.ds(k * 16, 16)]
            return 0
        lax.fori_loop(0, 32, radd, 0)
    pltpu.sync_copy(acc2, shfin.at[pl.ds(2 * s, 2)])
    plsc.subcore_barrier()
    pltpu.sync_copy(shfin, dn)

    def wcomp(i, _):
        off = c * HALF + i * 16
        dv = dstb[pl.ds(off, 16)]
        for p in range(4):
            ev = eb[p, pl.ds(off, 16)]
            g = plsc.load_gather(dn, [(dv >> 9) + 8 * p, dv & 511])
            wb_[p, pl.ds(i * 16, 16)] = ev / (g + EPS)
        return 0
    lax.fori_loop(0, HALF // 16, wcomp, 0)

    for p in range(4):
        pltpu.sync_copy(wb_.at[p], out_hbm.at[p, pl.ds(base + c * HALF, HALF)])


WROWS = 16
NWIN = N_NODES // WROWS
WPT = NWIN // NS
CAP = 16384
SCHUNK = 1024


@functools.partial(
    pl.kernel,
    out_type=jax.ShapeDtypeStruct((4 * N_NODES, N_NODES), jnp.float32),
    mesh=_sc_mesh,
    scratch_types=[
        pltpu.VMEM((SCHUNK,), jnp.int32),
        pltpu.VMEM((SCHUNK,), jnp.int32),
        pltpu.VMEM((SCHUNK,), jnp.float32),
        pltpu.VMEM((SCHUNK,), jnp.float32),
        pltpu.VMEM((CAP,), jnp.int32),
        pltpu.VMEM((CAP,), jnp.float32),
        pltpu.VMEM((CAP,), jnp.float32),
        pltpu.VMEM((WROWS, N_NODES), jnp.float32),
    ],
    compiler_params=_sc_params,
)
def _build_dense(src_hbm, dst_hbm, w_hbm, dense_hbm,
                 ssrc, sdst, sw0, sw1, kk, kw0, kw1, slab):
    c = lax.axis_index("c")
    s = lax.axis_index("s")

    def stream(i, cnt):
        off = i * SCHUNK
        pltpu.sync_copy(src_hbm.at[pl.ds(off, SCHUNK)], ssrc)
        pltpu.sync_copy(dst_hbm.at[pl.ds(off, SCHUNK)], sdst)
        pltpu.sync_copy(w_hbm.at[2 * c, pl.ds(off, SCHUNK)], sw0)
        pltpu.sync_copy(w_hbm.at[2 * c + 1, pl.ds(off, SCHUNK)], sw1)

        def chunk(j, cnt2):
            o2 = j * 16
            sv = ssrc[pl.ds(o2, 16)]
            dv = sdst[pl.ds(o2, 16)]
            key = sv * 4096 + dv
            m = ((sv >> 4) & 15) == s
            cc = jnp.minimum(cnt2, CAP - 16)
            plsc.store_compressed(kk.at[pl.ds(cc, 16)], key, mask=m)
            plsc.store_compressed(kw0.at[pl.ds(cc, 16)], sw0[pl.ds(o2, 16)], mask=m)
            plsc.store_compressed(kw1.at[pl.ds(cc, 16)], sw1[pl.ds(o2, 16)], mask=m)
            return cc + jnp.sum(m.astype(jnp.int32))
        return lax.fori_loop(0, SCHUNK // 16, chunk, cnt)
    cnt = lax.fori_loop(0, N_EDGES // SCHUNK, stream, 0)
    nchunks = (cnt + 15) >> 4

    def sweep(k, _):
        ph = k >> 4
        plane = c * 2 + ph
        wid = (k & 15) * NS + s
        row0 = wid * WROWS

        def zero_row(i, _):
            for k2 in range(16):
                slab[i >> 4, pl.ds(((i & 15) * 16 + k2) * 16, 16)] = (
                    jnp.zeros((16,), jnp.float32))
            return 0
        lax.fori_loop(0, WROWS * 16, zero_row, 0)

        def scan(i, _):
            o2 = i * 16
            kv = kk[pl.ds(o2, 16)]
            m = (kv >> 16) == wid
            wv = jnp.where(ph == 0, kw0[pl.ds(o2, 16)], kw1[pl.ds(o2, 16)])
            wv = jnp.where(m, wv, 0.0)
            plsc.addupdate_scatter(slab, [(kv >> 12) & 15, kv & 4095], wv)
            return 0
        lax.fori_loop(0, nchunks, scan, 0)

        pltpu.sync_copy(slab, dense_hbm.at[pl.ds(plane * N_NODES + row0, WROWS)])
        return 0

    lax.fori_loop(0, 2 * WPT, sweep, 0)


BM = 512
BN = 2048


def _mm_body(a_ref, b_ref, o_ref):
    o_ref[0] = jnp.dot(a_ref[0], b_ref[0], preferred_element_type=jnp.float32)


@jax.jit
def _mm(ab):
    n = ab.shape[-1]
    grid = (2, n // BN, n // BM)
    return pl.pallas_call(
        _mm_body,
        grid=grid,
        in_specs=[
            pl.BlockSpec((1, BM, n), lambda p, j, i: (p, i, 0)),
            pl.BlockSpec((1, n, BN), lambda p, j, i: (p + 2, 0, j)),
        ],
        out_specs=pl.BlockSpec((1, BM, BN), lambda p, j, i: (p, i, j)),
        out_shape=jax.ShapeDtypeStruct((2, n, n), jnp.float32),
        compiler_params=pltpu.CompilerParams(
            dimension_semantics=("arbitrary", "arbitrary", "arbitrary"),
        ),
    )(ab, ab)


def kernel(edge_w, W1, W2, edge_src, edge_dst):
    Wc = jnp.concatenate([W1, W2], axis=0).reshape(16)
    Wbrd = jnp.broadcast_to(Wc[:, None], (16, 16))
    w = _edge_norm(edge_w, Wbrd, edge_dst)
    dense = _build_dense(edge_src, edge_dst, w).reshape(4, N_NODES, N_NODES)
    C = _mm(dense.astype(jnp.bfloat16))
    return (C, W1, W2, w[:2])
